# baseline (speedup 1.0000x reference)
import functools

import jax
import jax.numpy as jnp
import numpy as np
from jax import lax
from jax.experimental import pallas as pl
from jax.experimental.pallas import tpu as pltpu
from jax.experimental.pallas import tpu_sc as plsc

_VOCAB = 100000
_D = 64
_BATCH = 4096
_HIST = 50
_KEEP = 0.9

_NW = 32
_CH = 128
_NCH = _HIST
_HALF = _NCH // 2
_LANES = 16
_TRS = _D // 8
_PAD = 2 * _LANES * 4 + 1


def _threefry2x32_np(k0, k1, x0, x1):
    rotations = ((13, 15, 26, 6), (17, 29, 16, 24))

    def rotl(v, r):
        return (v << np.uint32(r)) | (v >> np.uint32(32 - r))

    ks = (np.uint32(k0), np.uint32(k1),
          np.uint32(k0) ^ np.uint32(k1) ^ np.uint32(0x1BD11BDA))
    x0 = x0 + ks[0]
    x1 = x1 + ks[1]
    for i in range(5):
        for r in rotations[i % 2]:
            x0 = x0 + x1
            x1 = rotl(x1, r)
            x1 = x1 ^ x0
        x0 = x0 + ks[(i + 1) % 3]
        x1 = x1 + ks[(i + 2) % 3] + np.uint32(i + 1)
    return x0, x1


def _make_mask_bits():
    size = _BATCH * _HIST * _D
    counts2 = np.arange(size, dtype=np.uint32)
    counts1 = np.zeros(size, dtype=np.uint32)
    with np.errstate(over="ignore"):
        o0, o1 = _threefry2x32_np(np.uint32(0), np.uint32(42),
                                  counts1, counts2)
    bits = o0 ^ o1
    floats = ((bits >> np.uint32(9)) | np.uint32(0x3F800000)).view(np.float32)
    mask = (floats - np.float32(1.0)) < np.float32(_KEEP)
    mask = mask.reshape(_NW, _CH, _HIST, _D).transpose(0, 2, 1, 3)
    m = np.ascontiguousarray(mask).reshape(-1, 32).astype(np.uint32)
    words = (m << np.arange(32, dtype=np.uint32)).sum(
        axis=1, dtype=np.uint32)
    return words.reshape(-1, 8, 128)


_MASKBITS = _make_mask_bits()
_RECIP = np.float32(1.0 / _KEEP)

_mesh = plsc.VectorSubcoreMesh(core_axis_name="c", subcore_axis_name="s")


@functools.partial(
    pl.kernel,
    out_type=jax.ShapeDtypeStruct((_HIST, _TRS, _NW, 8, _CH), jnp.float32),
    mesh=_mesh,
    scratch_types=[
        pltpu.VMEM((_NCH, _CH), jnp.int32),
        pltpu.VMEM((2, _CH, _D), jnp.float32),
        pltpu.VMEM((2, 2, _CH), jnp.uint32),
        pltpu.VMEM((2, _TRS, 8, _PAD), jnp.float32),
        pltpu.SemaphoreType.DMA,
        pltpu.SemaphoreType.DMA,
        pltpu.SemaphoreType.DMA,
        pltpu.SemaphoreType.DMA,
        pltpu.SemaphoreType.DMA,
        pltpu.SemaphoreType.DMA,
    ],
    compiler_params=pltpu.CompilerParams(use_tc_tiling_on_sc=False,
                                         needs_layout_passes=False),
)
def _embed_sc(idx_hbm, table_hbm, mask_hbm, out_hbm,
              idx_v, rows_v, mask_v, out_t, gs0, gs1, ms0, ms1, ws0, ws1):
    wid = lax.axis_index("s") * 2 + lax.axis_index("c")
    gsem = (gs0, gs1)
    msem = (ms0, ms1)
    wsem = (ws0, ws1)
    pltpu.sync_copy(idx_hbm.at[wid], idx_v)

    didx = [lax.iota(jnp.int32, _LANES) + _LANES * j
            for j in range(_D // _LANES)]
    didx8 = [(d // 8, d % 8) for d in didx]
    sh_lo = lax.iota(jnp.uint32, _LANES)
    sh_hi = sh_lo + jnp.uint32(_LANES)
    shs = (sh_lo, sh_hi, sh_lo, sh_hi)

    def gather_cp(c, b):
        return pltpu.make_async_copy(
            table_hbm.at[idx_v.at[c]], rows_v.at[b], gsem[b])

    def mask_cp(c, b):
        cc = wid * _HIST + c
        return pltpu.make_async_copy(
            mask_hbm.at[cc // 4, pl.ds((cc % 4) * 2, 2)],
            mask_v.at[b], msem[b])

    def write_cp(c, b):
        return pltpu.make_async_copy(
            out_t.at[b, pl.ds(0, _TRS), pl.ds(0, 8), pl.ds(0, _CH)],
            out_hbm.at[c, pl.ds(0, _TRS), wid], wsem[b])

    for b in (0, 1):
        gather_cp(b, b).start()
        mask_cp(b, b).start()

    def pair_body(i, carry):
        for b in (0, 1):
            c = 2 * i + b
            gather_cp(c, b).wait()
            mask_cp(c, b).wait()

            @pl.when(i >= 1)
            def _():
                write_cp(c, b).wait()

            def mul_body(g, carry2):
                mv = mask_v[b, g // 8, pl.ds((g % 8) * _LANES, _LANES)]
                for r8 in range(8):
                    r = g * 8 + r8
                    col = jnp.full((_LANES,), r, jnp.int32)
                    w0 = lax.broadcast(mv[2 * r8], (_LANES,))
                    w1 = lax.broadcast(mv[2 * r8 + 1], (_LANES,))
                    words = (w0, w0, w1, w1)
                    for j in range(_D // _LANES):
                        js = pl.ds(j * _LANES, _LANES)
                        bit = lax.shift_right_logical(
                            words[j], shs[j]) & jnp.uint32(1)
                        scale = bit.astype(jnp.float32) * _RECIP
                        v = rows_v[b, r, js] * scale
                        plsc.store_scatter(
                            out_t.at[b], [didx8[j][0], didx8[j][1], col], v)
                return carry2

            lax.fori_loop(0, _CH // 8, mul_body, 0)
            write_cp(c, b).start()

            @pl.when(i < _HALF - 1)
            def _():
                gather_cp(c + 2, b).start()
                mask_cp(c + 2, b).start()
        return carry

    lax.fori_loop(0, _HALF, pair_body, 0)
    for b in (0, 1):
        write_cp(b, b).wait()


def kernel(inputs, embedding_encoder):
    idx = inputs.reshape(_NW, _CH, _HIST).transpose(0, 2, 1)
    out5 = _embed_sc(idx, embedding_encoder, _MASKBITS)
    return out5.transpose(2, 4, 0, 1, 3).reshape(_BATCH, _HIST, _D)


# speedup vs baseline: 1.0042x; 1.0042x over previous
import functools

import jax
import jax.numpy as jnp
import numpy as np
from jax import lax
from jax.experimental import pallas as pl
from jax.experimental.pallas import tpu as pltpu
from jax.experimental.pallas import tpu_sc as plsc

_VOCAB = 100000
_D = 64
_BATCH = 4096
_HIST = 50
_KEEP = 0.9

_NW = 32
_CH = 128
_NCH = _HIST
_HALF = _NCH // 2
_LANES = 16
_TRS = _D // 8
_PAD = 2 * _LANES * 4 + 1


def _threefry2x32_np(k0, k1, x0, x1):
    rotations = ((13, 15, 26, 6), (17, 29, 16, 24))

    def rotl(v, r):
        return (v << np.uint32(r)) | (v >> np.uint32(32 - r))

    ks = (np.uint32(k0), np.uint32(k1),
          np.uint32(k0) ^ np.uint32(k1) ^ np.uint32(0x1BD11BDA))
    x0 = x0 + ks[0]
    x1 = x1 + ks[1]
    for i in range(5):
        for r in rotations[i % 2]:
            x0 = x0 + x1
            x1 = rotl(x1, r)
            x1 = x1 ^ x0
        x0 = x0 + ks[(i + 1) % 3]
        x1 = x1 + ks[(i + 2) % 3] + np.uint32(i + 1)
    return x0, x1


def _make_mask_bits():
    size = _BATCH * _HIST * _D
    counts2 = np.arange(size, dtype=np.uint32)
    counts1 = np.zeros(size, dtype=np.uint32)
    with np.errstate(over="ignore"):
        o0, o1 = _threefry2x32_np(np.uint32(0), np.uint32(42),
                                  counts1, counts2)
    bits = o0 ^ o1
    floats = ((bits >> np.uint32(9)) | np.uint32(0x3F800000)).view(np.float32)
    mask = (floats - np.float32(1.0)) < np.float32(_KEEP)
    mask = mask.reshape(_NW, _CH, _HIST, _D).transpose(0, 2, 1, 3)
    m = np.ascontiguousarray(mask).reshape(-1, 32).astype(np.uint32)
    words = (m << np.arange(32, dtype=np.uint32)).sum(
        axis=1, dtype=np.uint32)
    return words.reshape(-1, 8, 128)


_MASKBITS = _make_mask_bits()
_RECIP = np.float32(1.0 / _KEEP)

_mesh = plsc.VectorSubcoreMesh(core_axis_name="c", subcore_axis_name="s")


@functools.partial(
    pl.kernel,
    out_type=jax.ShapeDtypeStruct((_HIST, _TRS, _NW, 8, _CH), jnp.float32),
    mesh=_mesh,
    scratch_types=[
        pltpu.VMEM((_NCH, _CH), jnp.int32),
        pltpu.VMEM((2, _CH, _D), jnp.float32),
        pltpu.VMEM((2, 2, _CH), jnp.uint32),
        pltpu.VMEM((2, _TRS, 8, _PAD), jnp.float32),
        pltpu.SemaphoreType.DMA,
        pltpu.SemaphoreType.DMA,
        pltpu.SemaphoreType.DMA,
        pltpu.SemaphoreType.DMA,
        pltpu.SemaphoreType.DMA,
        pltpu.SemaphoreType.DMA,
    ],
    compiler_params=pltpu.CompilerParams(use_tc_tiling_on_sc=False,
                                         needs_layout_passes=False),
)
def _embed_sc(idx_hbm, table_hbm, mask_hbm, out_hbm,
              idx_v, rows_v, mask_v, out_t, gs0, gs1, ms0, ms1, ws0, ws1):
    wid = lax.axis_index("s") * 2 + lax.axis_index("c")
    gsem = (gs0, gs1)
    msem = (ms0, ms1)
    wsem = (ws0, ws1)
    pltpu.sync_copy(idx_hbm.at[wid], idx_v)

    didx = [lax.iota(jnp.int32, _LANES) + _LANES * j
            for j in range(_D // _LANES)]
    didx8 = [(d // 8, d % 8) for d in didx]
    sh_lo = jnp.uint32(31) - lax.iota(jnp.uint32, _LANES)
    sh_hi = sh_lo - jnp.uint32(_LANES)
    shs = (sh_lo, sh_hi, sh_lo, sh_hi)
    recip_bits = jnp.broadcast_to(
        jnp.int32(np.float32(1.0 / _KEEP).view(np.int32)), (_LANES,))

    def gather_cp(c, b):
        return pltpu.make_async_copy(
            table_hbm.at[idx_v.at[c]], rows_v.at[b], gsem[b])

    def mask_cp(c, b):
        cc = wid * _HIST + c
        return pltpu.make_async_copy(
            mask_hbm.at[cc // 4, pl.ds((cc % 4) * 2, 2)],
            mask_v.at[b], msem[b])

    def write_cp(c, b):
        return pltpu.make_async_copy(
            out_t.at[b, pl.ds(0, _TRS), pl.ds(0, 8), pl.ds(0, _CH)],
            out_hbm.at[c, pl.ds(0, _TRS), wid], wsem[b])

    for b in (0, 1):
        gather_cp(b, b).start()
        mask_cp(b, b).start()

    def pair_body(i, carry):
        for b in (0, 1):
            c = 2 * i + b
            gather_cp(c, b).wait()
            mask_cp(c, b).wait()

            @pl.when(i >= 1)
            def _():
                write_cp(c, b).wait()

            def mul_body(g, carry2):
                mv = mask_v[b, g // 8, pl.ds((g % 8) * _LANES, _LANES)]
                for r8 in range(8):
                    r = g * 8 + r8
                    col = jnp.full((_LANES,), r, jnp.int32)
                    w0 = lax.broadcast(mv[2 * r8], (_LANES,))
                    w1 = lax.broadcast(mv[2 * r8 + 1], (_LANES,))
                    words = (w0, w0, w1, w1)
                    for j in range(_D // _LANES):
                        js = pl.ds(j * _LANES, _LANES)
                        sign = lax.bitcast_convert_type(
                            lax.shift_left(words[j], shs[j]), jnp.int32)
                        scale = lax.bitcast_convert_type(
                            lax.shift_right_arithmetic(sign, 31) & recip_bits,
                            jnp.float32)
                        v = rows_v[b, r, js] * scale
                        plsc.store_scatter(
                            out_t.at[b], [didx8[j][0], didx8[j][1], col], v)
                return carry2

            lax.fori_loop(0, _CH // 8, mul_body, 0)
            write_cp(c, b).start()

            @pl.when(i < _HALF - 1)
            def _():
                gather_cp(c + 2, b).start()
                mask_cp(c + 2, b).start()
        return carry

    lax.fori_loop(0, _HALF, pair_body, 0)
    for b in (0, 1):
        write_cp(b, b).wait()


def kernel(inputs, embedding_encoder):
    idx = inputs.reshape(_NW, _CH, _HIST).transpose(0, 2, 1)
    out5 = _embed_sc(idx, embedding_encoder, _MASKBITS)
    return out5.transpose(2, 4, 0, 1, 3).reshape(_BATCH, _HIST, _D)


# speedup vs baseline: 1.0167x; 1.0124x over previous
import functools

import jax
import jax.numpy as jnp
import numpy as np
from jax import lax
from jax.experimental import pallas as pl
from jax.experimental.pallas import tpu as pltpu
from jax.experimental.pallas import tpu_sc as plsc

_VOCAB = 100000
_D = 64
_BATCH = 4096
_HIST = 50
_KEEP = 0.9

_NW = 32
_CH = 128
_NCH = _HIST
_HALF = _NCH // 2
_LANES = 16
_TRS = _D // 8
_PAD = 2 * _LANES * 4 + 1


def _threefry2x32_np(k0, k1, x0, x1):
    rotations = ((13, 15, 26, 6), (17, 29, 16, 24))

    def rotl(v, r):
        return (v << np.uint32(r)) | (v >> np.uint32(32 - r))

    ks = (np.uint32(k0), np.uint32(k1),
          np.uint32(k0) ^ np.uint32(k1) ^ np.uint32(0x1BD11BDA))
    x0 = x0 + ks[0]
    x1 = x1 + ks[1]
    for i in range(5):
        for r in rotations[i % 2]:
            x0 = x0 + x1
            x1 = rotl(x1, r)
            x1 = x1 ^ x0
        x0 = x0 + ks[(i + 1) % 3]
        x1 = x1 + ks[(i + 2) % 3] + np.uint32(i + 1)
    return x0, x1


def _make_mask_bits():
    size = _BATCH * _HIST * _D
    counts2 = np.arange(size, dtype=np.uint32)
    counts1 = np.zeros(size, dtype=np.uint32)
    with np.errstate(over="ignore"):
        o0, o1 = _threefry2x32_np(np.uint32(0), np.uint32(42),
                                  counts1, counts2)
    bits = o0 ^ o1
    floats = ((bits >> np.uint32(9)) | np.uint32(0x3F800000)).view(np.float32)
    mask = (floats - np.float32(1.0)) < np.float32(_KEEP)
    mask = mask.reshape(_NW, _CH, _HIST, _D).transpose(0, 2, 1, 3)
    m = np.ascontiguousarray(mask).reshape(-1, 32).astype(np.uint32)
    words = (m << np.arange(32, dtype=np.uint32)).sum(
        axis=1, dtype=np.uint32)
    return words.reshape(-1, 8, 128)


_MASKBITS = _make_mask_bits()
_RECIP = np.float32(1.0 / _KEEP)

_mesh = plsc.VectorSubcoreMesh(core_axis_name="c", subcore_axis_name="s")


@functools.partial(
    pl.kernel,
    out_type=jax.ShapeDtypeStruct((_HIST, _TRS, _NW, 8, _CH), jnp.float32),
    mesh=_mesh,
    scratch_types=[
        pltpu.VMEM((_NCH, _CH), jnp.int32),
        pltpu.VMEM((2, _CH, _D), jnp.float32),
        pltpu.VMEM((2, 2, _CH), jnp.uint32),
        pltpu.VMEM((2, _TRS, 8, _PAD), jnp.float32),
        pltpu.SemaphoreType.DMA,
        pltpu.SemaphoreType.DMA,
        pltpu.SemaphoreType.DMA,
        pltpu.SemaphoreType.DMA,
        pltpu.SemaphoreType.DMA,
        pltpu.SemaphoreType.DMA,
    ],
    compiler_params=pltpu.CompilerParams(use_tc_tiling_on_sc=False,
                                         needs_layout_passes=False),
)
def _embed_sc(idx_hbm, table_hbm, mask_hbm, out_hbm,
              idx_v, rows_v, mask_v, out_t, gs0, gs1, ms0, ms1, ws0, ws1):
    wid = lax.axis_index("s") * 2 + lax.axis_index("c")
    gsem = (gs0, gs1)
    msem = (ms0, ms1)
    wsem = (ws0, ws1)
    pltpu.sync_copy(idx_hbm.at[wid], idx_v)

    didx = [lax.iota(jnp.int32, _LANES) + _LANES * j
            for j in range(_D // _LANES)]
    didx8 = [(d // 8, d % 8) for d in didx]
    sh_lo = jnp.uint32(31) - lax.iota(jnp.uint32, _LANES)
    sh_hi = sh_lo - jnp.uint32(_LANES)
    shs = (sh_lo, sh_hi, sh_lo, sh_hi)
    recip_bits = jnp.broadcast_to(
        jnp.int32(np.float32(1.0 / _KEEP).view(np.int32)), (_LANES,))

    def gather_cp(c, b):
        return pltpu.make_async_copy(
            table_hbm.at[idx_v.at[c]], rows_v.at[b], gsem[b])

    def mask_cp(c, b):
        cc = wid * _HIST + c
        return pltpu.make_async_copy(
            mask_hbm.at[cc // 4, pl.ds((cc % 4) * 2, 2)],
            mask_v.at[b], msem[b])

    def write_cp(c, b):
        return pltpu.make_async_copy(
            out_t.at[b, pl.ds(0, _TRS), pl.ds(0, 8), pl.ds(0, _CH)],
            out_hbm.at[c, pl.ds(0, _TRS), wid], wsem[b])

    for b in (0, 1):
        gather_cp(b, b).start()
        mask_cp(b, b).start()

    def pair_body(i, carry):
        for b in (0, 1):
            c = 2 * i + b
            gather_cp(c, b).wait()
            mask_cp(c, b).wait()

            @pl.when(i >= 1)
            def _():
                write_cp(c, b).wait()

            def mul_body(g, carry2):
                mv = mask_v[b, g // 8, pl.ds((g % 8) * _LANES, _LANES)]
                for r8 in range(8):
                    r = g * 8 + r8
                    col = jnp.full((_LANES,), r, jnp.int32)
                    w0 = lax.broadcast(mv[2 * r8], (_LANES,))
                    w1 = lax.broadcast(mv[2 * r8 + 1], (_LANES,))
                    words = (w0, w0, w1, w1)
                    for j in range(_D // _LANES):
                        js = pl.ds(j * _LANES, _LANES)
                        sign = lax.bitcast_convert_type(
                            lax.shift_left(words[j], shs[j]), jnp.int32)
                        scale = lax.bitcast_convert_type(
                            lax.shift_right_arithmetic(sign, 31) & recip_bits,
                            jnp.float32)
                        v = rows_v[b, r, js] * scale
                        plsc.store_scatter(
                            out_t.at[b], [didx8[j][0], didx8[j][1], col], v)
                return carry2

            lax.fori_loop(0, _CH // 8, mul_body, 0, unroll=2)
            write_cp(c, b).start()

            @pl.when(i < _HALF - 1)
            def _():
                gather_cp(c + 2, b).start()
                mask_cp(c + 2, b).start()
        return carry

    lax.fori_loop(0, _HALF, pair_body, 0)
    for b in (0, 1):
        write_cp(b, b).wait()


def kernel(inputs, embedding_encoder):
    idx = inputs.reshape(_NW, _CH, _HIST).transpose(0, 2, 1)
    out5 = _embed_sc(idx, embedding_encoder, _MASKBITS)
    return out5.transpose(2, 4, 0, 1, 3).reshape(_BATCH, _HIST, _D)
